# baseline (speedup 1.0000x reference)
import functools

import jax
import jax.numpy as jnp
from jax import lax
from jax.experimental import pallas as pl
from jax.experimental.pallas import tpu as pltpu
from jax.experimental.pallas import tpu_sc as plsc

N = 10000
E = 320000
D_IN = 518
D_NODE = 128
NUM_REL = 26

NC = 2
NS = 16
NW = NC * NS
K2 = 128
GPW = 80
EPW = GPW * K2
EPAD = NW * EPW
DW = 128
NZ = 624
ZR = 8
CB = 16

NB = 2000
GRID = N // NB


def _tc_a_body(x_ref, w1_ref, b1_ref, w2_ref, b2_ref, lw_ref, asrc_ref,
               adst_ref, hp_ref, hpb_ref, asad_ref):
    h = jnp.maximum(
        jnp.dot(x_ref[...], w1_ref[...], preferred_element_type=jnp.float32)
        + b1_ref[...], 0.0)
    h = jnp.dot(h, w2_ref[...], preferred_element_type=jnp.float32) + b2_ref[...]
    hp = jnp.dot(h, lw_ref[...], preferred_element_type=jnp.float32)
    hp_ref[...] = hp
    hpb_ref[...] = hp.astype(jnp.bfloat16)
    a_s = jnp.sum(hp * asrc_ref[...], axis=1)
    a_d = jnp.sum(hp * adst_ref[...], axis=1)
    asad_ref[...] = jnp.stack([a_s, a_d], axis=1)


def _tc_a(x, w1, b1, w2, b2, lw, asrc, adst):
    return pl.pallas_call(
        _tc_a_body,
        grid=(GRID,),
        in_specs=[
            pl.BlockSpec((NB, D_IN), lambda i: (i, 0)),
            pl.BlockSpec((D_IN, 256), lambda i: (0, 0)),
            pl.BlockSpec((1, 256), lambda i: (0, 0)),
            pl.BlockSpec((256, D_NODE), lambda i: (0, 0)),
            pl.BlockSpec((1, D_NODE), lambda i: (0, 0)),
            pl.BlockSpec((D_NODE, D_NODE), lambda i: (0, 0)),
            pl.BlockSpec((1, D_NODE), lambda i: (0, 0)),
            pl.BlockSpec((1, D_NODE), lambda i: (0, 0)),
        ],
        out_specs=[
            pl.BlockSpec((NB, D_NODE), lambda i: (i, 0)),
            pl.BlockSpec((NB, D_NODE), lambda i: (i, 0)),
            pl.BlockSpec((NB, 2), lambda i: (i, 0)),
        ],
        out_shape=[
            jax.ShapeDtypeStruct((N, D_NODE), jnp.float32),
            jax.ShapeDtypeStruct((N, D_NODE), jnp.bfloat16),
            jax.ShapeDtypeStruct((N, 2), jnp.float32),
        ],
    )(x, w1, b1, w2, b2, lw, asrc, adst)


def _combine(num_ref, den_ref, hp_ref, asad_ref, aloop_ref, bias_ref):
    aloop = aloop_ref[0, 0]
    asv = asad_ref[...]
    al = asv[:, 0] + asv[:, 1] + aloop
    al = jnp.where(al >= 0.0, al, 0.2 * al)
    exl = jnp.exp(al)
    v = num_ref[...]
    num = v[0] + v[1] + exl[:, None] * hp_ref[...]
    den = jnp.sum(den_ref[...], axis=1, keepdims=True) + exl[:, None] + 1e-16
    return jnp.maximum(num / den + bias_ref[...], 0.0)


def _tc_b_body(num_ref, den_ref, hp_ref, asad_ref, aloop_ref, bias_ref, lw_ref,
               asrc_ref, adst_ref, hp2_ref, hp2b_ref, asad2_ref):
    h2 = _combine(num_ref, den_ref, hp_ref, asad_ref, aloop_ref, bias_ref)
    hp2 = jnp.dot(h2, lw_ref[...], preferred_element_type=jnp.float32)
    hp2_ref[...] = hp2
    hp2b_ref[...] = hp2.astype(jnp.bfloat16)
    a_s = jnp.sum(hp2 * asrc_ref[...], axis=1)
    a_d = jnp.sum(hp2 * adst_ref[...], axis=1)
    asad2_ref[...] = jnp.stack([a_s, a_d], axis=1)


def _tc_b(num, den, hp, asad, aloop, bias, lw, asrc, adst):
    return pl.pallas_call(
        _tc_b_body,
        grid=(GRID,),
        in_specs=[
            pl.BlockSpec((2, NB, DW), lambda i: (0, i, 0)),
            pl.BlockSpec((NB, NW), lambda i: (i, 0)),
            pl.BlockSpec((NB, D_NODE), lambda i: (i, 0)),
            pl.BlockSpec((NB, 2), lambda i: (i, 0)),
            pl.BlockSpec((1, 1), lambda i: (0, 0), memory_space=pltpu.SMEM),
            pl.BlockSpec((1, D_NODE), lambda i: (0, 0)),
            pl.BlockSpec((D_NODE, D_NODE), lambda i: (0, 0)),
            pl.BlockSpec((1, D_NODE), lambda i: (0, 0)),
            pl.BlockSpec((1, D_NODE), lambda i: (0, 0)),
        ],
        out_specs=[
            pl.BlockSpec((NB, D_NODE), lambda i: (i, 0)),
            pl.BlockSpec((NB, D_NODE), lambda i: (i, 0)),
            pl.BlockSpec((NB, 2), lambda i: (i, 0)),
        ],
        out_shape=[
            jax.ShapeDtypeStruct((N, D_NODE), jnp.float32),
            jax.ShapeDtypeStruct((N, D_NODE), jnp.bfloat16),
            jax.ShapeDtypeStruct((N, 2), jnp.float32),
        ],
    )(num, den, hp, asad, aloop, bias, lw, asrc, adst)


def _tc_c_body(num_ref, den_ref, hp_ref, asad_ref, aloop_ref, bias_ref, ow1_ref,
               ob1_ref, ow2_ref, ob2_ref, out_ref, acc_ref):
    i = pl.program_id(0)
    h3 = _combine(num_ref, den_ref, hp_ref, asad_ref, aloop_ref, bias_ref)
    psum = jnp.sum(h3, axis=0, keepdims=True)

    @pl.when(i == 0)
    def _():
        acc_ref[...] = psum

    @pl.when(i > 0)
    def _():
        acc_ref[...] = acc_ref[...] + psum

    @pl.when(i == pl.num_programs(0) - 1)
    def _():
        g = acc_ref[...] * (1.0 / N)
        o1 = jnp.maximum(
            jnp.dot(g, ow1_ref[...], preferred_element_type=jnp.float32)
            + ob1_ref[...], 0.0)
        out_ref[...] = jnp.dot(
            o1, ow2_ref[...], preferred_element_type=jnp.float32) + ob2_ref[...]


def _tc_c(num, den, hp, asad, aloop, bias, ow1, ob1, ow2, ob2):
    return pl.pallas_call(
        _tc_c_body,
        grid=(GRID,),
        in_specs=[
            pl.BlockSpec((2, NB, DW), lambda i: (0, i, 0)),
            pl.BlockSpec((NB, NW), lambda i: (i, 0)),
            pl.BlockSpec((NB, D_NODE), lambda i: (i, 0)),
            pl.BlockSpec((NB, 2), lambda i: (i, 0)),
            pl.BlockSpec((1, 1), lambda i: (0, 0), memory_space=pltpu.SMEM),
            pl.BlockSpec((1, D_NODE), lambda i: (0, 0)),
            pl.BlockSpec((D_NODE, 64), lambda i: (0, 0)),
            pl.BlockSpec((1, 64), lambda i: (0, 0)),
            pl.BlockSpec((64, 32), lambda i: (0, 0)),
            pl.BlockSpec((1, 32), lambda i: (0, 0)),
        ],
        out_specs=pl.BlockSpec((1, 32), lambda i: (0, 0)),
        out_shape=jax.ShapeDtypeStruct((1, 32), jnp.float32),
        scratch_shapes=[pltpu.VMEM((1, D_NODE), jnp.float32)],
    )(num, den, hp, asad, aloop, bias, ow1, ob1, ow2, ob2)


def _sc_alpha_body(src_h, dst_h, attr_h, asad_h, atab_a_h, atab_b_h,
                   ex_o, den_o, aes_o,
                   src_v, dst_v, attr_v, asad_v, ta_v, tb_v, ex_v, den_v,
                   stage_v):
    cid = lax.axis_index("c")
    sid = lax.axis_index("s")
    wid = cid * NS + sid

    pltpu.sync_copy(src_h.at[wid], src_v)
    pltpu.sync_copy(dst_h.at[wid], dst_v)
    pltpu.sync_copy(attr_h.at[wid], attr_v)
    pltpu.sync_copy(asad_h, asad_v)
    pltpu.sync_copy(atab_a_h, ta_v)
    pltpu.sync_copy(atab_b_h, tb_v)

    z16f = jnp.zeros((16,), jnp.float32)

    def _zden(i, c):
        den_v[pl.ds(i * 16, 16)] = z16f
        return c
    lax.fori_loop(0, N // 16, _zden, 0)

    oo16 = jnp.ones((16,), jnp.int32)
    lanes = lax.iota(jnp.int32, 16)
    masks = [lanes == t for t in range(16)]
    zf = jnp.float32(0.0)
    ebase = wid * EPW

    def _alpha(g, carry):
        acc1, acc2 = carry
        for c in range(K2 // 16):
            sl = pl.ds(c * 16, 16)
            sv = src_v[g, sl]
            dv = dst_v[g, sl]
            av = attr_v[g, sl]
            valid = (ebase + g * K2 + c * 16 + lanes) < E
            a_s = plsc.load_gather(asad_v, [sv + sv])
            a_d = plsc.load_gather(asad_v, [dv + dv + oo16])
            t1 = plsc.load_gather(ta_v, [av])
            t2 = plsc.load_gather(tb_v, [av])
            acc1 = acc1 + jnp.where(valid, t1, zf)
            acc2 = acc2 + jnp.where(valid, t2, zf)
            al = a_s + a_d + t1
            al = jnp.where(al >= 0.0, al, 0.2 * al)
            ex = jnp.where(valid, jnp.exp(al), zf)
            ex_v[g, sl] = ex
            for t in range(16):
                plsc.addupdate_scatter(den_v, [dv], ex, mask=masks[t])
        return acc1, acc2

    acc1, acc2 = lax.fori_loop(0, GPW, _alpha, (z16f, z16f))
    stage_v[0, pl.ds(0, 16)] = acc1
    stage_v[1, pl.ds(0, 16)] = acc2
    pltpu.sync_copy(stage_v, aes_o.at[wid])
    pltpu.sync_copy(ex_v, ex_o.at[wid])
    dwb = pl.multiple_of(wid * N, 8)
    pltpu.sync_copy(den_v, den_o.at[pl.ds(dwb, N)])


_sc_alpha = pl.kernel(
    _sc_alpha_body,
    out_type=(
        jax.ShapeDtypeStruct((NW, GPW, K2), jnp.float32),
        jax.ShapeDtypeStruct((NW * N,), jnp.float32),
        jax.ShapeDtypeStruct((NW, 2, 16), jnp.float32),
    ),
    mesh=plsc.VectorSubcoreMesh(core_axis_name="c", subcore_axis_name="s"),
    compiler_params=pltpu.CompilerParams(needs_layout_passes=False),
    scratch_types=[
        pltpu.VMEM((GPW, K2), jnp.int32),
        pltpu.VMEM((GPW, K2), jnp.int32),
        pltpu.VMEM((GPW, K2), jnp.int32),
        pltpu.VMEM((2 * N,), jnp.float32),
        pltpu.VMEM((32,), jnp.float32),
        pltpu.VMEM((32,), jnp.float32),
        pltpu.VMEM((GPW, K2), jnp.float32),
        pltpu.VMEM((N,), jnp.float32),
        pltpu.VMEM((2, 16), jnp.float32),
    ],
)


def _sc_rows_body(src_h, dst_h, ex_h, hp_h,
                  num_o,
                  srcb, dstb, exb, gbuf0, gbuf1, zer_v, num_s,
                  sem_g0, sem_g1, sem_s0, sem_s1):
    cid = lax.axis_index("c")
    sid = lax.axis_index("s")
    wid = cid * NS + sid

    z16f = jnp.zeros((16,), jnp.float32)

    def _zrow(i, c):
        for cc in range(DW // 16):
            zer_v[i, pl.ds(cc * 16, 16)] = z16f
        return c
    lax.fori_loop(0, ZR, _zrow, 0)

    zbase = pl.multiple_of(sid * NZ, 8)

    def _zcp(j, c):
        pltpu.sync_copy(zer_v, num_s.at[pl.ds(zbase + j * ZR, ZR)])
        return c
    lax.fori_loop(0, NZ // ZR, _zcp, 0)

    @pl.when(sid == NS - 1)
    def _():
        for j in range((N - NS * NZ) // ZR):
            pltpu.sync_copy(zer_v, num_s.at[pl.ds(NS * NZ + j * ZR, ZR)])
    plsc.subcore_barrier()

    def _scale(buf, l):
        @plsc.parallel_loop(0, K2 // 16, 1, unroll=2)
        def _scale16(jj):
            exv = exb[l, pl.ds(jj * 16, 16)]
            for t in range(16):
                j = jj * 16 + t
                ex_s = exv[t]
                for cc in range(D_NODE // 16):
                    sl = pl.ds(cc * 16, 16)
                    buf[j, sl] = buf[j, sl] * ex_s

    SG = 4
    SR = K2 // SG

    def _gstart(buf, sem, l):
        for q in range(SG):
            pltpu.async_copy(hp_h.at[srcb.at[l, pl.ds(q * SR, SR)]],
                             buf.at[pl.ds(q * SR, SR)], sem)

    def _gwait(buf, sem, l):
        for q in range(SG):
            pltpu.make_async_copy(hp_h.at[srcb.at[l, pl.ds(q * SR, SR)]],
                                  buf.at[pl.ds(q * SR, SR)], sem).wait()

    def _block(b, c):
        row0 = pl.multiple_of(b * CB, 8)
        pltpu.sync_copy(src_h.at[wid, pl.ds(row0, CB)], srcb)
        pltpu.sync_copy(dst_h.at[wid, pl.ds(row0, CB)], dstb)
        pltpu.sync_copy(ex_h.at[wid, pl.ds(row0, CB)], exb)
        _gstart(gbuf0, sem_g0, 0)
        _gstart(gbuf1, sem_g1, 1)

        def _pair(k, c2):
            l0 = 2 * k
            l1 = 2 * k + 1
            _gwait(gbuf0, sem_g0, l0)
            _scale(gbuf0, l0)
            sc0 = pltpu.async_copy(gbuf0, num_s.at[dstb.at[l0]], sem_s0,
                                   add=True)
            _gwait(gbuf1, sem_g1, l1)
            _scale(gbuf1, l1)
            sc1 = pltpu.async_copy(gbuf1, num_s.at[dstb.at[l1]], sem_s1,
                                   add=True)
            sc0.wait()
            _gstart(gbuf0, sem_g0, l0 + 2)
            sc1.wait()
            _gstart(gbuf1, sem_g1, l1 + 2)
            return c2
        lax.fori_loop(0, CB // 2 - 1, _pair, 0)

        _gwait(gbuf0, sem_g0, CB - 2)
        _scale(gbuf0, CB - 2)
        pltpu.async_copy(gbuf0, num_s.at[dstb.at[CB - 2]], sem_s0, add=True)
        _gwait(gbuf1, sem_g1, CB - 1)
        _scale(gbuf1, CB - 1)
        pltpu.async_copy(gbuf1, num_s.at[dstb.at[CB - 1]], sem_s1, add=True)
        pltpu.make_async_copy(gbuf0, num_s.at[dstb.at[CB - 2]], sem_s0).wait()
        pltpu.make_async_copy(gbuf1, num_s.at[dstb.at[CB - 1]], sem_s1).wait()
        return c
    lax.fori_loop(0, GPW // CB, _block, 0)

    plsc.subcore_barrier()
    dbase = pl.multiple_of(sid * NZ, 8)
    pltpu.sync_copy(num_s.at[pl.ds(dbase, NZ)],
                    num_o.at[cid, pl.ds(dbase, NZ)])

    @pl.when(sid == NS - 1)
    def _():
        pltpu.sync_copy(num_s.at[pl.ds(NS * NZ, N - NS * NZ)],
                        num_o.at[cid, pl.ds(NS * NZ, N - NS * NZ)])


_sc_rows = pl.kernel(
    _sc_rows_body,
    out_type=jax.ShapeDtypeStruct((NC, N, DW), jnp.float32),
    mesh=plsc.VectorSubcoreMesh(core_axis_name="c", subcore_axis_name="s"),
    compiler_params=pltpu.CompilerParams(needs_layout_passes=False),
    scratch_types=[
        pltpu.VMEM((CB, K2), jnp.int32),
        pltpu.VMEM((CB, K2), jnp.int32),
        pltpu.VMEM((CB, K2), jnp.float32),
        pltpu.VMEM((K2, D_NODE), jnp.float32),
        pltpu.VMEM((K2, D_NODE), jnp.float32),
        pltpu.VMEM((ZR, DW), jnp.float32),
        pltpu.VMEM_SHARED((N, DW), jnp.float32),
        pltpu.SemaphoreType.DMA,
        pltpu.SemaphoreType.DMA,
        pltpu.SemaphoreType.DMA,
        pltpu.SemaphoreType.DMA,
    ],
)


def kernel(x, edge_index, edge_attr, params):
    p = params
    c1, c2 = p['conv1'], p['conv2']

    atab1 = p['rel_emb'] @ (c1['lin_edge_W'] @ c1['att_edge'])
    atab2 = p['rel_emb'] @ (c2['lin_edge_W'] @ c2['att_edge'])
    atab1 = jnp.pad(atab1, (0, 32 - NUM_REL))
    atab2 = jnp.pad(atab2, (0, 32 - NUM_REL))
    pad = EPAD - E
    src2 = jnp.pad(edge_index[0], (0, pad)).reshape(NW, GPW, K2)
    dst2 = jnp.pad(edge_index[1], (0, pad)).reshape(NW, GPW, K2)
    attr2 = jnp.pad(edge_attr, (0, pad)).reshape(NW, GPW, K2)

    def row2d(v):
        return v.reshape(1, -1)

    hp1, hp1b, asad1 = _tc_a(x, p['enc_W1'], row2d(p['enc_b1']), p['enc_W2'],
                       row2d(p['enc_b2']), c1['lin_W'], row2d(c1['att_src']),
                       row2d(c1['att_dst']))

    ex1, den1, aes = _sc_alpha(src2, dst2, attr2, asad1.reshape(-1), atab1,
                               atab2)
    num1 = _sc_rows(src2, dst2, ex1, hp1)
    aloop1 = (jnp.sum(aes[:, 0, :]) / E).reshape(1, 1)
    aloop2 = (jnp.sum(aes[:, 1, :]) / E).reshape(1, 1)

    hp2, hp2b, asad2 = _tc_b(num1, den1.reshape(NW, N).T, hp1, asad1, aloop1,
                       row2d(c1['bias']), c2['lin_W'], row2d(c2['att_src']),
                       row2d(c2['att_dst']))

    ex2, den2, _ = _sc_alpha(src2, dst2, attr2, asad2.reshape(-1), atab2,
                             atab1)
    num2 = _sc_rows(src2, dst2, ex2, hp2)

    return _tc_c(num2, den2.reshape(NW, N).T, hp2, asad2, aloop2,
                 row2d(c2['bias']), p['out_W1'], row2d(p['out_b1']),
                 p['out_W2'], row2d(p['out_b2']))


# speedup vs baseline: 1.0355x; 1.0355x over previous
import functools

import jax
import jax.numpy as jnp
from jax import lax
from jax.experimental import pallas as pl
from jax.experimental.pallas import tpu as pltpu
from jax.experimental.pallas import tpu_sc as plsc

N = 10000
E = 320000
D_IN = 518
D_NODE = 128
NUM_REL = 26

NC = 2
NS = 16
NW = NC * NS
K2 = 128
GPW = 80
EPW = GPW * K2
EPAD = NW * EPW
DW = 128
NZ = 624
ZR = 8
CB = 16

NB = 2000
GRID = N // NB


def _tc_a_body(x_ref, w1_ref, b1_ref, w2_ref, b2_ref, lw_ref, asrc_ref,
               adst_ref, hp_ref, hpb_ref, asad_ref):
    h = jnp.maximum(
        jnp.dot(x_ref[...], w1_ref[...], preferred_element_type=jnp.float32)
        + b1_ref[...], 0.0)
    h = jnp.dot(h, w2_ref[...], preferred_element_type=jnp.float32) + b2_ref[...]
    hp = jnp.dot(h, lw_ref[...], preferred_element_type=jnp.float32)
    hp_ref[...] = hp
    hpb_ref[...] = hp.astype(jnp.bfloat16)
    a_s = jnp.sum(hp * asrc_ref[...], axis=1)
    a_d = jnp.sum(hp * adst_ref[...], axis=1)
    asad_ref[...] = jnp.stack([a_s, a_d], axis=1)


def _tc_a(x, w1, b1, w2, b2, lw, asrc, adst):
    return pl.pallas_call(
        _tc_a_body,
        grid=(GRID,),
        in_specs=[
            pl.BlockSpec((NB, D_IN), lambda i: (i, 0)),
            pl.BlockSpec((D_IN, 256), lambda i: (0, 0)),
            pl.BlockSpec((1, 256), lambda i: (0, 0)),
            pl.BlockSpec((256, D_NODE), lambda i: (0, 0)),
            pl.BlockSpec((1, D_NODE), lambda i: (0, 0)),
            pl.BlockSpec((D_NODE, D_NODE), lambda i: (0, 0)),
            pl.BlockSpec((1, D_NODE), lambda i: (0, 0)),
            pl.BlockSpec((1, D_NODE), lambda i: (0, 0)),
        ],
        out_specs=[
            pl.BlockSpec((NB, D_NODE), lambda i: (i, 0)),
            pl.BlockSpec((NB, D_NODE), lambda i: (i, 0)),
            pl.BlockSpec((NB, 2), lambda i: (i, 0)),
        ],
        out_shape=[
            jax.ShapeDtypeStruct((N, D_NODE), jnp.float32),
            jax.ShapeDtypeStruct((N, D_NODE), jnp.bfloat16),
            jax.ShapeDtypeStruct((N, 2), jnp.float32),
        ],
    )(x, w1, b1, w2, b2, lw, asrc, adst)


def _combine(num_ref, den_ref, hp_ref, asad_ref, aloop_ref, bias_ref):
    aloop = aloop_ref[0, 0]
    asv = asad_ref[...]
    al = asv[:, 0] + asv[:, 1] + aloop
    al = jnp.where(al >= 0.0, al, 0.2 * al)
    exl = jnp.exp(al)
    v = num_ref[...]
    num = v[0] + v[1] + exl[:, None] * hp_ref[...]
    den = jnp.sum(den_ref[...], axis=1, keepdims=True) + exl[:, None] + 1e-16
    return jnp.maximum(num / den + bias_ref[...], 0.0)


def _tc_b_body(num_ref, den_ref, hp_ref, asad_ref, aloop_ref, bias_ref, lw_ref,
               asrc_ref, adst_ref, hp2_ref, hp2b_ref, asad2_ref):
    h2 = _combine(num_ref, den_ref, hp_ref, asad_ref, aloop_ref, bias_ref)
    hp2 = jnp.dot(h2, lw_ref[...], preferred_element_type=jnp.float32)
    hp2_ref[...] = hp2
    hp2b_ref[...] = hp2.astype(jnp.bfloat16)
    a_s = jnp.sum(hp2 * asrc_ref[...], axis=1)
    a_d = jnp.sum(hp2 * adst_ref[...], axis=1)
    asad2_ref[...] = jnp.stack([a_s, a_d], axis=1)


def _tc_b(num, den, hp, asad, aloop, bias, lw, asrc, adst):
    return pl.pallas_call(
        _tc_b_body,
        grid=(GRID,),
        in_specs=[
            pl.BlockSpec((2, NB, DW), lambda i: (0, i, 0)),
            pl.BlockSpec((NB, NW), lambda i: (i, 0)),
            pl.BlockSpec((NB, D_NODE), lambda i: (i, 0)),
            pl.BlockSpec((NB, 2), lambda i: (i, 0)),
            pl.BlockSpec((1, 1), lambda i: (0, 0), memory_space=pltpu.SMEM),
            pl.BlockSpec((1, D_NODE), lambda i: (0, 0)),
            pl.BlockSpec((D_NODE, D_NODE), lambda i: (0, 0)),
            pl.BlockSpec((1, D_NODE), lambda i: (0, 0)),
            pl.BlockSpec((1, D_NODE), lambda i: (0, 0)),
        ],
        out_specs=[
            pl.BlockSpec((NB, D_NODE), lambda i: (i, 0)),
            pl.BlockSpec((NB, D_NODE), lambda i: (i, 0)),
            pl.BlockSpec((NB, 2), lambda i: (i, 0)),
        ],
        out_shape=[
            jax.ShapeDtypeStruct((N, D_NODE), jnp.float32),
            jax.ShapeDtypeStruct((N, D_NODE), jnp.bfloat16),
            jax.ShapeDtypeStruct((N, 2), jnp.float32),
        ],
    )(num, den, hp, asad, aloop, bias, lw, asrc, adst)


def _tc_c_body(num_ref, den_ref, hp_ref, asad_ref, aloop_ref, bias_ref, ow1_ref,
               ob1_ref, ow2_ref, ob2_ref, out_ref, acc_ref):
    i = pl.program_id(0)
    h3 = _combine(num_ref, den_ref, hp_ref, asad_ref, aloop_ref, bias_ref)
    psum = jnp.sum(h3, axis=0, keepdims=True)

    @pl.when(i == 0)
    def _():
        acc_ref[...] = psum

    @pl.when(i > 0)
    def _():
        acc_ref[...] = acc_ref[...] + psum

    @pl.when(i == pl.num_programs(0) - 1)
    def _():
        g = acc_ref[...] * (1.0 / N)
        o1 = jnp.maximum(
            jnp.dot(g, ow1_ref[...], preferred_element_type=jnp.float32)
            + ob1_ref[...], 0.0)
        out_ref[...] = jnp.dot(
            o1, ow2_ref[...], preferred_element_type=jnp.float32) + ob2_ref[...]


def _tc_c(num, den, hp, asad, aloop, bias, ow1, ob1, ow2, ob2):
    return pl.pallas_call(
        _tc_c_body,
        grid=(GRID,),
        in_specs=[
            pl.BlockSpec((2, NB, DW), lambda i: (0, i, 0)),
            pl.BlockSpec((NB, NW), lambda i: (i, 0)),
            pl.BlockSpec((NB, D_NODE), lambda i: (i, 0)),
            pl.BlockSpec((NB, 2), lambda i: (i, 0)),
            pl.BlockSpec((1, 1), lambda i: (0, 0), memory_space=pltpu.SMEM),
            pl.BlockSpec((1, D_NODE), lambda i: (0, 0)),
            pl.BlockSpec((D_NODE, 64), lambda i: (0, 0)),
            pl.BlockSpec((1, 64), lambda i: (0, 0)),
            pl.BlockSpec((64, 32), lambda i: (0, 0)),
            pl.BlockSpec((1, 32), lambda i: (0, 0)),
        ],
        out_specs=pl.BlockSpec((1, 32), lambda i: (0, 0)),
        out_shape=jax.ShapeDtypeStruct((1, 32), jnp.float32),
        scratch_shapes=[pltpu.VMEM((1, D_NODE), jnp.float32)],
    )(num, den, hp, asad, aloop, bias, ow1, ob1, ow2, ob2)


def _sc_alpha_body(src_h, dst_h, attr_h, asad_h, atab_a_h, atab_b_h,
                   ex_o, den_o, aes_o,
                   src_v, dst_v, attr_v, asad_v, ta_v, tb_v, ex_v, den_v,
                   stage_v):
    cid = lax.axis_index("c")
    sid = lax.axis_index("s")
    wid = cid * NS + sid

    pltpu.sync_copy(src_h.at[wid], src_v)
    pltpu.sync_copy(dst_h.at[wid], dst_v)
    pltpu.sync_copy(attr_h.at[wid], attr_v)
    pltpu.sync_copy(asad_h, asad_v)
    pltpu.sync_copy(atab_a_h, ta_v)
    pltpu.sync_copy(atab_b_h, tb_v)

    z16f = jnp.zeros((16,), jnp.float32)

    def _zden(i, c):
        den_v[pl.ds(i * 16, 16)] = z16f
        return c
    lax.fori_loop(0, N // 16, _zden, 0)

    oo16 = jnp.ones((16,), jnp.int32)
    lanes = lax.iota(jnp.int32, 16)
    masks = [lanes == t for t in range(16)]
    zf = jnp.float32(0.0)
    ebase = wid * EPW

    def _alpha(g, carry):
        acc1, acc2 = carry
        for c in range(K2 // 16):
            sl = pl.ds(c * 16, 16)
            sv = src_v[g, sl]
            dv = dst_v[g, sl]
            av = attr_v[g, sl]
            valid = (ebase + g * K2 + c * 16 + lanes) < E
            a_s = plsc.load_gather(asad_v, [sv + sv])
            a_d = plsc.load_gather(asad_v, [dv + dv + oo16])
            t1 = plsc.load_gather(ta_v, [av])
            t2 = plsc.load_gather(tb_v, [av])
            acc1 = acc1 + jnp.where(valid, t1, zf)
            acc2 = acc2 + jnp.where(valid, t2, zf)
            al = a_s + a_d + t1
            al = jnp.where(al >= 0.0, al, 0.2 * al)
            ex = jnp.where(valid, jnp.exp(al), zf)
            ex_v[g, sl] = ex
            for t in range(16):
                plsc.addupdate_scatter(den_v, [dv], ex, mask=masks[t])
        return acc1, acc2

    acc1, acc2 = lax.fori_loop(0, GPW, _alpha, (z16f, z16f))
    stage_v[0, pl.ds(0, 16)] = acc1
    stage_v[1, pl.ds(0, 16)] = acc2
    pltpu.sync_copy(stage_v, aes_o.at[wid])
    pltpu.sync_copy(ex_v, ex_o.at[wid])
    dwb = pl.multiple_of(wid * N, 8)
    pltpu.sync_copy(den_v, den_o.at[pl.ds(dwb, N)])


_sc_alpha = pl.kernel(
    _sc_alpha_body,
    out_type=(
        jax.ShapeDtypeStruct((NW, GPW, K2), jnp.float32),
        jax.ShapeDtypeStruct((NW * N,), jnp.float32),
        jax.ShapeDtypeStruct((NW, 2, 16), jnp.float32),
    ),
    mesh=plsc.VectorSubcoreMesh(core_axis_name="c", subcore_axis_name="s"),
    compiler_params=pltpu.CompilerParams(needs_layout_passes=False),
    scratch_types=[
        pltpu.VMEM((GPW, K2), jnp.int32),
        pltpu.VMEM((GPW, K2), jnp.int32),
        pltpu.VMEM((GPW, K2), jnp.int32),
        pltpu.VMEM((2 * N,), jnp.float32),
        pltpu.VMEM((32,), jnp.float32),
        pltpu.VMEM((32,), jnp.float32),
        pltpu.VMEM((GPW, K2), jnp.float32),
        pltpu.VMEM((N,), jnp.float32),
        pltpu.VMEM((2, 16), jnp.float32),
    ],
)


def _sc_rows_body(src_h, dst_h, ex_h, hp_h,
                  num_o,
                  srcb, dstb, exb, gbuf0, gbuf1, zer_v, num_s,
                  sem_g0, sem_g1, sem_s0, sem_s1):
    cid = lax.axis_index("c")
    sid = lax.axis_index("s")
    wid = cid * NS + sid

    z16f = jnp.zeros((16,), jnp.float32)

    def _zrow(i, c):
        for cc in range(DW // 16):
            zer_v[i, pl.ds(cc * 16, 16)] = z16f
        return c
    lax.fori_loop(0, ZR, _zrow, 0)

    zbase = pl.multiple_of(sid * NZ, 8)

    def _zcp(j, c):
        pltpu.sync_copy(zer_v, num_s.at[pl.ds(zbase + j * ZR, ZR)])
        return c
    lax.fori_loop(0, NZ // ZR, _zcp, 0)

    @pl.when(sid == NS - 1)
    def _():
        for j in range((N - NS * NZ) // ZR):
            pltpu.sync_copy(zer_v, num_s.at[pl.ds(NS * NZ + j * ZR, ZR)])
    plsc.subcore_barrier()

    def _scale(buf, l):
        @plsc.parallel_loop(0, K2 // 16, 1, unroll=2)
        def _scale16(jj):
            exv = exb[l, pl.ds(jj * 16, 16)]
            for t in range(16):
                j = jj * 16 + t
                ex_s = exv[t]
                for cc in range(D_NODE // 16):
                    sl = pl.ds(cc * 16, 16)
                    buf[j, sl] = buf[j, sl] * ex_s

    SG = 4
    SR = K2 // SG

    def _gstart(buf, sem, l):
        for q in range(SG):
            pltpu.async_copy(hp_h.at[srcb.at[l, pl.ds(q * SR, SR)]],
                             buf.at[pl.ds(q * SR, SR)], sem)

    def _gwait(buf, sem, l):
        for q in range(SG):
            pltpu.make_async_copy(hp_h.at[srcb.at[l, pl.ds(q * SR, SR)]],
                                  buf.at[pl.ds(q * SR, SR)], sem).wait()

    def _block(b, c):
        row0 = pl.multiple_of(b * CB, 8)
        pltpu.sync_copy(src_h.at[wid, pl.ds(row0, CB)], srcb)
        pltpu.sync_copy(dst_h.at[wid, pl.ds(row0, CB)], dstb)
        pltpu.sync_copy(ex_h.at[wid, pl.ds(row0, CB)], exb)
        _gstart(gbuf0, sem_g0, 0)
        _gstart(gbuf1, sem_g1, 1)

        def _pair(k, c2):
            l0 = 2 * k
            l1 = 2 * k + 1
            _gwait(gbuf0, sem_g0, l0)
            _scale(gbuf0, l0)
            pltpu.sync_copy(gbuf0, num_s.at[dstb.at[l0]], add=True)
            _gstart(gbuf0, sem_g0, l0 + 2)
            _gwait(gbuf1, sem_g1, l1)
            _scale(gbuf1, l1)
            pltpu.sync_copy(gbuf1, num_s.at[dstb.at[l1]], add=True)
            _gstart(gbuf1, sem_g1, l1 + 2)
            return c2
        lax.fori_loop(0, CB // 2 - 1, _pair, 0)

        _gwait(gbuf0, sem_g0, CB - 2)
        _scale(gbuf0, CB - 2)
        pltpu.sync_copy(gbuf0, num_s.at[dstb.at[CB - 2]], add=True)
        _gwait(gbuf1, sem_g1, CB - 1)
        _scale(gbuf1, CB - 1)
        pltpu.sync_copy(gbuf1, num_s.at[dstb.at[CB - 1]], add=True)
        return c
    lax.fori_loop(0, GPW // CB, _block, 0)

    plsc.subcore_barrier()
    dbase = pl.multiple_of(sid * NZ, 8)
    pltpu.sync_copy(num_s.at[pl.ds(dbase, NZ)],
                    num_o.at[cid, pl.ds(dbase, NZ)])

    @pl.when(sid == NS - 1)
    def _():
        pltpu.sync_copy(num_s.at[pl.ds(NS * NZ, N - NS * NZ)],
                        num_o.at[cid, pl.ds(NS * NZ, N - NS * NZ)])


_sc_rows = pl.kernel(
    _sc_rows_body,
    out_type=jax.ShapeDtypeStruct((NC, N, DW), jnp.float32),
    mesh=plsc.VectorSubcoreMesh(core_axis_name="c", subcore_axis_name="s"),
    compiler_params=pltpu.CompilerParams(needs_layout_passes=False),
    scratch_types=[
        pltpu.VMEM((CB, K2), jnp.int32),
        pltpu.VMEM((CB, K2), jnp.int32),
        pltpu.VMEM((CB, K2), jnp.float32),
        pltpu.VMEM((K2, D_NODE), jnp.float32),
        pltpu.VMEM((K2, D_NODE), jnp.float32),
        pltpu.VMEM((ZR, DW), jnp.float32),
        pltpu.VMEM_SHARED((N, DW), jnp.float32),
        pltpu.SemaphoreType.DMA,
        pltpu.SemaphoreType.DMA,
        pltpu.SemaphoreType.DMA,
        pltpu.SemaphoreType.DMA,
    ],
)


def kernel(x, edge_index, edge_attr, params):
    p = params
    c1, c2 = p['conv1'], p['conv2']

    atab1 = p['rel_emb'] @ (c1['lin_edge_W'] @ c1['att_edge'])
    atab2 = p['rel_emb'] @ (c2['lin_edge_W'] @ c2['att_edge'])
    atab1 = jnp.pad(atab1, (0, 32 - NUM_REL))
    atab2 = jnp.pad(atab2, (0, 32 - NUM_REL))
    pad = EPAD - E
    src2 = jnp.pad(edge_index[0], (0, pad)).reshape(NW, GPW, K2)
    dst2 = jnp.pad(edge_index[1], (0, pad)).reshape(NW, GPW, K2)
    attr2 = jnp.pad(edge_attr, (0, pad)).reshape(NW, GPW, K2)

    def row2d(v):
        return v.reshape(1, -1)

    hp1, hp1b, asad1 = _tc_a(x, p['enc_W1'], row2d(p['enc_b1']), p['enc_W2'],
                       row2d(p['enc_b2']), c1['lin_W'], row2d(c1['att_src']),
                       row2d(c1['att_dst']))

    ex1, den1, aes = _sc_alpha(src2, dst2, attr2, asad1.reshape(-1), atab1,
                               atab2)
    num1 = _sc_rows(src2, dst2, ex1, hp1)
    aloop1 = (jnp.sum(aes[:, 0, :]) / E).reshape(1, 1)
    aloop2 = (jnp.sum(aes[:, 1, :]) / E).reshape(1, 1)

    hp2, hp2b, asad2 = _tc_b(num1, den1.reshape(NW, N).T, hp1, asad1, aloop1,
                       row2d(c1['bias']), c2['lin_W'], row2d(c2['att_src']),
                       row2d(c2['att_dst']))

    ex2, den2, _ = _sc_alpha(src2, dst2, attr2, asad2.reshape(-1), atab2,
                             atab1)
    num2 = _sc_rows(src2, dst2, ex2, hp2)

    return _tc_c(num2, den2.reshape(NW, N).T, hp2, asad2, aloop2,
                 row2d(c2['bias']), p['out_W1'], row2d(p['out_b1']),
                 p['out_W2'], row2d(p['out_b2']))


# speedup vs baseline: 1.0552x; 1.0190x over previous
import functools

import jax
import jax.numpy as jnp
from jax import lax
from jax.experimental import pallas as pl
from jax.experimental.pallas import tpu as pltpu
from jax.experimental.pallas import tpu_sc as plsc

N = 10000
E = 320000
D_IN = 518
D_NODE = 128
NUM_REL = 26

NC = 2
NS = 16
NW = NC * NS
K2 = 128
GPW = 80
EPW = GPW * K2
EPAD = NW * EPW
DW = 128
NZ = 624
ZR = 8
CB = 40

NB = 2000
GRID = N // NB


def _tc_a_body(x_ref, w1_ref, b1_ref, w2_ref, b2_ref, lw_ref, asrc_ref,
               adst_ref, hp_ref, hpb_ref, asad_ref):
    h = jnp.maximum(
        jnp.dot(x_ref[...], w1_ref[...], preferred_element_type=jnp.float32)
        + b1_ref[...], 0.0)
    h = jnp.dot(h, w2_ref[...], preferred_element_type=jnp.float32) + b2_ref[...]
    hp = jnp.dot(h, lw_ref[...], preferred_element_type=jnp.float32)
    hp_ref[...] = hp
    hpb_ref[...] = hp.astype(jnp.bfloat16)
    a_s = jnp.sum(hp * asrc_ref[...], axis=1)
    a_d = jnp.sum(hp * adst_ref[...], axis=1)
    asad_ref[...] = jnp.stack([a_s, a_d], axis=1)


def _tc_a(x, w1, b1, w2, b2, lw, asrc, adst):
    return pl.pallas_call(
        _tc_a_body,
        grid=(GRID,),
        in_specs=[
            pl.BlockSpec((NB, D_IN), lambda i: (i, 0)),
            pl.BlockSpec((D_IN, 256), lambda i: (0, 0)),
            pl.BlockSpec((1, 256), lambda i: (0, 0)),
            pl.BlockSpec((256, D_NODE), lambda i: (0, 0)),
            pl.BlockSpec((1, D_NODE), lambda i: (0, 0)),
            pl.BlockSpec((D_NODE, D_NODE), lambda i: (0, 0)),
            pl.BlockSpec((1, D_NODE), lambda i: (0, 0)),
            pl.BlockSpec((1, D_NODE), lambda i: (0, 0)),
        ],
        out_specs=[
            pl.BlockSpec((NB, D_NODE), lambda i: (i, 0)),
            pl.BlockSpec((NB, D_NODE), lambda i: (i, 0)),
            pl.BlockSpec((NB, 2), lambda i: (i, 0)),
        ],
        out_shape=[
            jax.ShapeDtypeStruct((N, D_NODE), jnp.float32),
            jax.ShapeDtypeStruct((N, D_NODE), jnp.bfloat16),
            jax.ShapeDtypeStruct((N, 2), jnp.float32),
        ],
    )(x, w1, b1, w2, b2, lw, asrc, adst)


def _combine(num_ref, den_ref, hp_ref, asad_ref, aloop_ref, bias_ref):
    aloop = aloop_ref[0, 0]
    asv = asad_ref[...]
    al = asv[:, 0] + asv[:, 1] + aloop
    al = jnp.where(al >= 0.0, al, 0.2 * al)
    exl = jnp.exp(al)
    v = num_ref[...]
    num = v[0] + v[1] + exl[:, None] * hp_ref[...]
    den = jnp.sum(den_ref[...], axis=1, keepdims=True) + exl[:, None] + 1e-16
    return jnp.maximum(num / den + bias_ref[...], 0.0)


def _tc_b_body(num_ref, den_ref, hp_ref, asad_ref, aloop_ref, bias_ref, lw_ref,
               asrc_ref, adst_ref, hp2_ref, hp2b_ref, asad2_ref):
    h2 = _combine(num_ref, den_ref, hp_ref, asad_ref, aloop_ref, bias_ref)
    hp2 = jnp.dot(h2, lw_ref[...], preferred_element_type=jnp.float32)
    hp2_ref[...] = hp2
    hp2b_ref[...] = hp2.astype(jnp.bfloat16)
    a_s = jnp.sum(hp2 * asrc_ref[...], axis=1)
    a_d = jnp.sum(hp2 * adst_ref[...], axis=1)
    asad2_ref[...] = jnp.stack([a_s, a_d], axis=1)


def _tc_b(num, den, hp, asad, aloop, bias, lw, asrc, adst):
    return pl.pallas_call(
        _tc_b_body,
        grid=(GRID,),
        in_specs=[
            pl.BlockSpec((2, NB, DW), lambda i: (0, i, 0)),
            pl.BlockSpec((NB, NW), lambda i: (i, 0)),
            pl.BlockSpec((NB, D_NODE), lambda i: (i, 0)),
            pl.BlockSpec((NB, 2), lambda i: (i, 0)),
            pl.BlockSpec((1, 1), lambda i: (0, 0), memory_space=pltpu.SMEM),
            pl.BlockSpec((1, D_NODE), lambda i: (0, 0)),
            pl.BlockSpec((D_NODE, D_NODE), lambda i: (0, 0)),
            pl.BlockSpec((1, D_NODE), lambda i: (0, 0)),
            pl.BlockSpec((1, D_NODE), lambda i: (0, 0)),
        ],
        out_specs=[
            pl.BlockSpec((NB, D_NODE), lambda i: (i, 0)),
            pl.BlockSpec((NB, D_NODE), lambda i: (i, 0)),
            pl.BlockSpec((NB, 2), lambda i: (i, 0)),
        ],
        out_shape=[
            jax.ShapeDtypeStruct((N, D_NODE), jnp.float32),
            jax.ShapeDtypeStruct((N, D_NODE), jnp.bfloat16),
            jax.ShapeDtypeStruct((N, 2), jnp.float32),
        ],
    )(num, den, hp, asad, aloop, bias, lw, asrc, adst)


def _tc_c_body(num_ref, den_ref, hp_ref, asad_ref, aloop_ref, bias_ref, ow1_ref,
               ob1_ref, ow2_ref, ob2_ref, out_ref, acc_ref):
    i = pl.program_id(0)
    h3 = _combine(num_ref, den_ref, hp_ref, asad_ref, aloop_ref, bias_ref)
    psum = jnp.sum(h3, axis=0, keepdims=True)

    @pl.when(i == 0)
    def _():
        acc_ref[...] = psum

    @pl.when(i > 0)
    def _():
        acc_ref[...] = acc_ref[...] + psum

    @pl.when(i == pl.num_programs(0) - 1)
    def _():
        g = acc_ref[...] * (1.0 / N)
        o1 = jnp.maximum(
            jnp.dot(g, ow1_ref[...], preferred_element_type=jnp.float32)
            + ob1_ref[...], 0.0)
        out_ref[...] = jnp.dot(
            o1, ow2_ref[...], preferred_element_type=jnp.float32) + ob2_ref[...]


def _tc_c(num, den, hp, asad, aloop, bias, ow1, ob1, ow2, ob2):
    return pl.pallas_call(
        _tc_c_body,
        grid=(GRID,),
        in_specs=[
            pl.BlockSpec((2, NB, DW), lambda i: (0, i, 0)),
            pl.BlockSpec((NB, NW), lambda i: (i, 0)),
            pl.BlockSpec((NB, D_NODE), lambda i: (i, 0)),
            pl.BlockSpec((NB, 2), lambda i: (i, 0)),
            pl.BlockSpec((1, 1), lambda i: (0, 0), memory_space=pltpu.SMEM),
            pl.BlockSpec((1, D_NODE), lambda i: (0, 0)),
            pl.BlockSpec((D_NODE, 64), lambda i: (0, 0)),
            pl.BlockSpec((1, 64), lambda i: (0, 0)),
            pl.BlockSpec((64, 32), lambda i: (0, 0)),
            pl.BlockSpec((1, 32), lambda i: (0, 0)),
        ],
        out_specs=pl.BlockSpec((1, 32), lambda i: (0, 0)),
        out_shape=jax.ShapeDtypeStruct((1, 32), jnp.float32),
        scratch_shapes=[pltpu.VMEM((1, D_NODE), jnp.float32)],
    )(num, den, hp, asad, aloop, bias, ow1, ob1, ow2, ob2)


def _sc_alpha_body(src_h, dst_h, attr_h, asad_h, atab_a_h, atab_b_h,
                   ex_o, den_o, aes_o,
                   src_v, dst_v, attr_v, asad_v, ta_v, tb_v, ex_v, den_v,
                   stage_v):
    cid = lax.axis_index("c")
    sid = lax.axis_index("s")
    wid = cid * NS + sid

    pltpu.sync_copy(src_h.at[wid], src_v)
    pltpu.sync_copy(dst_h.at[wid], dst_v)
    pltpu.sync_copy(attr_h.at[wid], attr_v)
    pltpu.sync_copy(asad_h, asad_v)
    pltpu.sync_copy(atab_a_h, ta_v)
    pltpu.sync_copy(atab_b_h, tb_v)

    z16f = jnp.zeros((16,), jnp.float32)

    def _zden(i, c):
        den_v[pl.ds(i * 16, 16)] = z16f
        return c
    lax.fori_loop(0, N // 16, _zden, 0)

    oo16 = jnp.ones((16,), jnp.int32)
    lanes = lax.iota(jnp.int32, 16)
    masks = [lanes == t for t in range(16)]
    zf = jnp.float32(0.0)
    ebase = wid * EPW

    def _alpha(g, carry):
        acc1, acc2 = carry
        for c in range(K2 // 16):
            sl = pl.ds(c * 16, 16)
            sv = src_v[g, sl]
            dv = dst_v[g, sl]
            av = attr_v[g, sl]
            valid = (ebase + g * K2 + c * 16 + lanes) < E
            a_s = plsc.load_gather(asad_v, [sv + sv])
            a_d = plsc.load_gather(asad_v, [dv + dv + oo16])
            t1 = plsc.load_gather(ta_v, [av])
            t2 = plsc.load_gather(tb_v, [av])
            acc1 = acc1 + jnp.where(valid, t1, zf)
            acc2 = acc2 + jnp.where(valid, t2, zf)
            al = a_s + a_d + t1
            al = jnp.where(al >= 0.0, al, 0.2 * al)
            ex = jnp.where(valid, jnp.exp(al), zf)
            ex_v[g, sl] = ex
            for t in range(16):
                plsc.addupdate_scatter(den_v, [dv], ex, mask=masks[t])
        return acc1, acc2

    acc1, acc2 = lax.fori_loop(0, GPW, _alpha, (z16f, z16f))
    stage_v[0, pl.ds(0, 16)] = acc1
    stage_v[1, pl.ds(0, 16)] = acc2
    pltpu.sync_copy(stage_v, aes_o.at[wid])
    pltpu.sync_copy(ex_v, ex_o.at[wid])
    dwb = pl.multiple_of(wid * N, 8)
    pltpu.sync_copy(den_v, den_o.at[pl.ds(dwb, N)])


_sc_alpha = pl.kernel(
    _sc_alpha_body,
    out_type=(
        jax.ShapeDtypeStruct((NW, GPW, K2), jnp.float32),
        jax.ShapeDtypeStruct((NW * N,), jnp.float32),
        jax.ShapeDtypeStruct((NW, 2, 16), jnp.float32),
    ),
    mesh=plsc.VectorSubcoreMesh(core_axis_name="c", subcore_axis_name="s"),
    compiler_params=pltpu.CompilerParams(needs_layout_passes=False),
    scratch_types=[
        pltpu.VMEM((GPW, K2), jnp.int32),
        pltpu.VMEM((GPW, K2), jnp.int32),
        pltpu.VMEM((GPW, K2), jnp.int32),
        pltpu.VMEM((2 * N,), jnp.float32),
        pltpu.VMEM((32,), jnp.float32),
        pltpu.VMEM((32,), jnp.float32),
        pltpu.VMEM((GPW, K2), jnp.float32),
        pltpu.VMEM((N,), jnp.float32),
        pltpu.VMEM((2, 16), jnp.float32),
    ],
)


def _sc_rows_body(src_h, dst_h, ex_h, hp_h,
                  num_o,
                  srcb, dstb, exb, gbuf0, gbuf1, zer_v, num_s,
                  sem_g0, sem_g1, sem_s0, sem_s1):
    cid = lax.axis_index("c")
    sid = lax.axis_index("s")
    wid = cid * NS + sid

    z16f = jnp.zeros((16,), jnp.float32)

    def _zrow(i, c):
        for cc in range(DW // 16):
            zer_v[i, pl.ds(cc * 16, 16)] = z16f
        return c
    lax.fori_loop(0, ZR, _zrow, 0)

    zbase = pl.multiple_of(sid * NZ, 8)

    def _zcp(j, c):
        pltpu.sync_copy(zer_v, num_s.at[pl.ds(zbase + j * ZR, ZR)])
        return c
    lax.fori_loop(0, NZ // ZR, _zcp, 0)

    @pl.when(sid == NS - 1)
    def _():
        for j in range((N - NS * NZ) // ZR):
            pltpu.sync_copy(zer_v, num_s.at[pl.ds(NS * NZ + j * ZR, ZR)])
    plsc.subcore_barrier()

    def _scale(buf, l):
        @plsc.parallel_loop(0, K2 // 16, 1, unroll=2)
        def _scale16(jj):
            exv = exb[l, pl.ds(jj * 16, 16)]
            for t in range(16):
                j = jj * 16 + t
                ex_s = exv[t]
                for cc in range(D_NODE // 16):
                    sl = pl.ds(cc * 16, 16)
                    buf[j, sl] = buf[j, sl] * ex_s

    SG = 4
    SR = K2 // SG

    def _gstart(buf, sem, l):
        for q in range(SG):
            pltpu.async_copy(hp_h.at[srcb.at[l, pl.ds(q * SR, SR)]],
                             buf.at[pl.ds(q * SR, SR)], sem)

    def _gwait(buf, sem, l):
        for q in range(SG):
            pltpu.make_async_copy(hp_h.at[srcb.at[l, pl.ds(q * SR, SR)]],
                                  buf.at[pl.ds(q * SR, SR)], sem).wait()

    def _block(b, c):
        row0 = pl.multiple_of(b * CB, 8)
        pltpu.sync_copy(src_h.at[wid, pl.ds(row0, CB)], srcb)
        pltpu.sync_copy(dst_h.at[wid, pl.ds(row0, CB)], dstb)
        pltpu.sync_copy(ex_h.at[wid, pl.ds(row0, CB)], exb)
        _gstart(gbuf0, sem_g0, 0)
        _gstart(gbuf1, sem_g1, 1)

        def _pair(k, c2):
            l0 = 2 * k
            l1 = 2 * k + 1
            _gwait(gbuf0, sem_g0, l0)
            _scale(gbuf0, l0)
            pltpu.sync_copy(gbuf0, num_s.at[dstb.at[l0]], add=True)
            _gstart(gbuf0, sem_g0, l0 + 2)
            _gwait(gbuf1, sem_g1, l1)
            _scale(gbuf1, l1)
            pltpu.sync_copy(gbuf1, num_s.at[dstb.at[l1]], add=True)
            _gstart(gbuf1, sem_g1, l1 + 2)
            return c2
        lax.fori_loop(0, CB // 2 - 1, _pair, 0)

        _gwait(gbuf0, sem_g0, CB - 2)
        _scale(gbuf0, CB - 2)
        pltpu.sync_copy(gbuf0, num_s.at[dstb.at[CB - 2]], add=True)
        _gwait(gbuf1, sem_g1, CB - 1)
        _scale(gbuf1, CB - 1)
        pltpu.sync_copy(gbuf1, num_s.at[dstb.at[CB - 1]], add=True)
        return c
    lax.fori_loop(0, GPW // CB, _block, 0)

    plsc.subcore_barrier()
    dbase = pl.multiple_of(sid * NZ, 8)
    pltpu.sync_copy(num_s.at[pl.ds(dbase, NZ)],
                    num_o.at[cid, pl.ds(dbase, NZ)])

    @pl.when(sid == NS - 1)
    def _():
        pltpu.sync_copy(num_s.at[pl.ds(NS * NZ, N - NS * NZ)],
                        num_o.at[cid, pl.ds(NS * NZ, N - NS * NZ)])


_sc_rows = pl.kernel(
    _sc_rows_body,
    out_type=jax.ShapeDtypeStruct((NC, N, DW), jnp.float32),
    mesh=plsc.VectorSubcoreMesh(core_axis_name="c", subcore_axis_name="s"),
    compiler_params=pltpu.CompilerParams(needs_layout_passes=False),
    scratch_types=[
        pltpu.VMEM((CB, K2), jnp.int32),
        pltpu.VMEM((CB, K2), jnp.int32),
        pltpu.VMEM((CB, K2), jnp.float32),
        pltpu.VMEM((K2, D_NODE), jnp.float32),
        pltpu.VMEM((K2, D_NODE), jnp.float32),
        pltpu.VMEM((ZR, DW), jnp.float32),
        pltpu.VMEM_SHARED((N, DW), jnp.float32),
        pltpu.SemaphoreType.DMA,
        pltpu.SemaphoreType.DMA,
        pltpu.SemaphoreType.DMA,
        pltpu.SemaphoreType.DMA,
    ],
)


def kernel(x, edge_index, edge_attr, params):
    p = params
    c1, c2 = p['conv1'], p['conv2']

    atab1 = p['rel_emb'] @ (c1['lin_edge_W'] @ c1['att_edge'])
    atab2 = p['rel_emb'] @ (c2['lin_edge_W'] @ c2['att_edge'])
    atab1 = jnp.pad(atab1, (0, 32 - NUM_REL))
    atab2 = jnp.pad(atab2, (0, 32 - NUM_REL))
    pad = EPAD - E
    src2 = jnp.pad(edge_index[0], (0, pad)).reshape(NW, GPW, K2)
    dst2 = jnp.pad(edge_index[1], (0, pad)).reshape(NW, GPW, K2)
    attr2 = jnp.pad(edge_attr, (0, pad)).reshape(NW, GPW, K2)

    def row2d(v):
        return v.reshape(1, -1)

    hp1, hp1b, asad1 = _tc_a(x, p['enc_W1'], row2d(p['enc_b1']), p['enc_W2'],
                       row2d(p['enc_b2']), c1['lin_W'], row2d(c1['att_src']),
                       row2d(c1['att_dst']))

    ex1, den1, aes = _sc_alpha(src2, dst2, attr2, asad1.reshape(-1), atab1,
                               atab2)
    num1 = _sc_rows(src2, dst2, ex1, hp1)
    aloop1 = (jnp.sum(aes[:, 0, :]) / E).reshape(1, 1)
    aloop2 = (jnp.sum(aes[:, 1, :]) / E).reshape(1, 1)

    hp2, hp2b, asad2 = _tc_b(num1, den1.reshape(NW, N).T, hp1, asad1, aloop1,
                       row2d(c1['bias']), c2['lin_W'], row2d(c2['att_src']),
                       row2d(c2['att_dst']))

    ex2, den2, _ = _sc_alpha(src2, dst2, attr2, asad2.reshape(-1), atab2,
                             atab1)
    num2 = _sc_rows(src2, dst2, ex2, hp2)

    return _tc_c(num2, den2.reshape(NW, N).T, hp2, asad2, aloop2,
                 row2d(c2['bias']), p['out_W1'], row2d(p['out_b1']),
                 p['out_W2'], row2d(p['out_b2']))
